# baseline (speedup 1.0000x reference)
import functools

import jax
import jax.numpy as jnp
from jax import lax
from jax.experimental import pallas as pl
from jax.experimental.pallas import tpu as pltpu
from jax.experimental.pallas import tpu_sc as plsc

N_ENV = 2
N = 10000
E2 = 160000
EPH = 163840
F = 64
NTILES = 16
K = 128
CPT = EPH // (NTILES * K)
MPT = 160
NPT = 624
BN = 1000
BE = 512

_f32 = jnp.float32
_INTERPRET = False


_sc_mesh = plsc.VectorSubcoreMesh(core_axis_name="c", subcore_axis_name="s",
                                  num_cores=2, num_subcores=NTILES)


@functools.partial(
    pl.kernel,
    out_type=jax.ShapeDtypeStruct((N_ENV * 2 * EPH, 2 * F), jnp.bfloat16),
    mesh=_sc_mesh,
    interpret=_INTERPRET,
    compiler_params=pltpu.CompilerParams(use_tc_tiling_on_sc=False),
    scratch_types=[
        pltpu.VMEM_SHARED((N, 2 * F), jnp.bfloat16),
        pltpu.VMEM((CPT, K), jnp.int32),
        pltpu.VMEM((CPT, K), jnp.int32),
        pltpu.VMEM((K, 2 * F), jnp.bfloat16),
        pltpu.VMEM((K, 2 * F), jnp.bfloat16),
        pltpu.VMEM((K, 2 * F), jnp.bfloat16),
        pltpu.VMEM((K, 2 * F), jnp.bfloat16),
        pltpu.VMEM((K, 2 * F), jnp.bfloat16),
        pltpu.VMEM((K, 2 * F), jnp.bfloat16),
        pltpu.SemaphoreType.DMA,
        pltpu.SemaphoreType.DMA,
        pltpu.SemaphoreType.DMA,
    ],
)
def _sc_gather(t_hbm, eab_hbm, g_hbm,
               t_sh, idxa_v, idxb_v, ra0, rb0, ra1, rb1, ra2, rb2,
               semA, semB, semC):
    c = lax.axis_index("c")
    s = lax.axis_index("s")
    row0 = pl.multiple_of(s * CPT, 8)
    pltpu.sync_copy(eab_hbm.at[0, pl.ds(row0, CPT)], idxa_v)
    pltpu.sync_copy(eab_hbm.at[1, pl.ds(row0, CPT)], idxb_v)

    stripe = pl.multiple_of(s * NPT, 8)
    base = pl.multiple_of(c * N, 8)
    pltpu.sync_copy(t_hbm.at[pl.ds(base + stripe, NPT)],
                    t_sh.at[pl.ds(stripe, NPT)])

    @pl.when(s == NTILES - 1)
    def _():
        tail = pl.multiple_of(NTILES * NPT, 8)
        pltpu.sync_copy(t_hbm.at[pl.ds(base + tail, N - NTILES * NPT)],
                        t_sh.at[pl.ds(tail, N - NTILES * NPT)])

    plsc.subcore_barrier()

    def issue(j, ra, rb, sem):
        pltpu.async_copy(t_sh.at[idxa_v.at[j]], ra, sem)
        pltpu.async_copy(t_sh.at[idxb_v.at[j]], rb, sem)

    def drain(j, ra, rb, sem):
        pltpu.make_async_copy(t_sh.at[idxa_v.at[j]], ra, sem).wait()
        pltpu.make_async_copy(t_sh.at[idxb_v.at[j]], rb, sem).wait()

    def store(j, ra, rb):
        out_row = pl.multiple_of(c * 2 * EPH + (row0 + j) * K, 8)
        pltpu.sync_copy(ra, g_hbm.at[pl.ds(out_row, K)])
        pltpu.sync_copy(rb, g_hbm.at[pl.ds(EPH + out_row, K)])

    bufs = ((ra0, rb0, semA), (ra1, rb1, semB), (ra2, rb2, semC))
    issue(0, *bufs[0])
    issue(1, *bufs[1])

    def step(j, ph):
        ra, rb, sem = bufs[ph]
        rn, rbn, semn = bufs[(ph + 2) % 3]

        @pl.when(j < CPT)
        def _():
            drain(j, ra, rb, sem)

            @pl.when(j + 2 < CPT)
            def _():
                issue(j + 2, rn, rbn, semn)

            store(j, ra, rb)

    def body(u, carry):
        j0 = 3 * u
        step(j0, 0)
        step(j0 + 1, 1)
        step(j0 + 2, 2)
        return carry

    lax.fori_loop(0, (CPT + 2) // 3, body, 0)


@functools.partial(
    pl.kernel,
    out_type=jax.ShapeDtypeStruct((N_ENV, N, F), _f32),
    mesh=_sc_mesh,
    interpret=_INTERPRET,
    compiler_params=pltpu.CompilerParams(use_tc_tiling_on_sc=False),
    scratch_types=[
        pltpu.VMEM_SHARED((N + 16, F), _f32),
        pltpu.VMEM((CPT, K), jnp.int32),
        pltpu.VMEM((CPT, K), jnp.int32),
        pltpu.VMEM((K, F), _f32),
        pltpu.VMEM((K, F), _f32),
        pltpu.SemaphoreType.DMA,
        pltpu.SemaphoreType.DMA,
    ],
)
def _sc_scatter(m_hbm, ef_hbm, eb_hbm, zeros_hbm, out_hbm, acc_sh, idxf_v,
                idxb_v, m0_v, m1_v, semA, semB):
    c = lax.axis_index("c")
    s = lax.axis_index("s")
    nrow0 = pl.multiple_of(s * NPT, 8)
    pltpu.sync_copy(zeros_hbm.at[pl.ds(nrow0, NPT)],
                    acc_sh.at[pl.ds(nrow0, NPT)])

    @pl.when(s == NTILES - 1)
    def _():
        tail = pl.multiple_of(NTILES * NPT, 8)
        pltpu.sync_copy(zeros_hbm.at[pl.ds(tail, N - NTILES * NPT)],
                        acc_sh.at[pl.ds(tail, N - NTILES * NPT)])

    row0 = pl.multiple_of(s * CPT, 8)
    pltpu.sync_copy(ef_hbm.at[pl.ds(row0, CPT)], idxf_v)
    pltpu.sync_copy(eb_hbm.at[pl.ds(row0, CPT)], idxb_v)
    plsc.subcore_barrier()

    def run_half(d, idx_v):
        def load(j, mv, sem):
            in_row = pl.multiple_of((row0 + j) * K, 8)
            pltpu.async_copy(m_hbm.at[c, d, pl.ds(in_row, K)], mv, sem)

        def drain(j, mv, sem):
            in_row = pl.multiple_of((row0 + j) * K, 8)
            pltpu.make_async_copy(m_hbm.at[c, d, pl.ds(in_row, K)],
                                  mv, sem).wait()

        load(0, m0_v, semA)

        def body(u, carry):
            j0 = 2 * u
            j1 = j0 + 1
            load(j1, m1_v, semB)
            drain(j0, m0_v, semA)
            pltpu.sync_copy(m0_v, acc_sh.at[idx_v.at[j0]], add=True)

            @pl.when(j1 + 1 < CPT)
            def _():
                load(j1 + 1, m0_v, semA)

            drain(j1, m1_v, semB)
            pltpu.sync_copy(m1_v, acc_sh.at[idx_v.at[j1]], add=True)
            return carry

        lax.fori_loop(0, CPT // 2, body, 0)

    run_half(0, idxf_v)
    run_half(1, idxb_v)
    plsc.subcore_barrier()
    pltpu.sync_copy(acc_sh.at[pl.ds(nrow0, NPT)],
                    out_hbm.at[c, pl.ds(nrow0, NPT)])

    @pl.when(s == NTILES - 1)
    def _():
        tail = pl.multiple_of(NTILES * NPT, 8)
        pltpu.sync_copy(acc_sh.at[pl.ds(tail, N - NTILES * NPT)],
                        out_hbm.at[c, pl.ds(tail, N - NTILES * NPT)])


def _pq_body(nf_ref, w1s_ref, w1d_ref, t_ref):
    x = nf_ref[0]
    p = jnp.dot(x, w1s_ref[...], preferred_element_type=_f32)
    q = jnp.dot(x, w1d_ref[...], preferred_element_type=_f32)
    t_ref[0] = jnp.concatenate([p, q], axis=1).astype(jnp.bfloat16)


def _tc_pq(nf, w1s, w1d):
    blk = lambda c, i: (c, i, 0)
    wspec = pl.BlockSpec((F, F), lambda c, i: (0, 0))
    return pl.pallas_call(
        _pq_body,
        grid=(N_ENV, N // BN),
        in_specs=[pl.BlockSpec((1, BN, F), blk), wspec, wspec],
        out_specs=pl.BlockSpec((1, BN, 2 * F), blk),
        out_shape=jax.ShapeDtypeStruct((N_ENV, N, 2 * F), jnp.bfloat16),
        interpret=_INTERPRET,
    )(nf, w1s, w1d)


def _mlp_body(gat_ref, gbt_ref, gab_ref, gbb_ref, b1_ref, w2_ref, b2_ref,
              w3_ref, b3_ref, m_ref):
    gat = gat_ref[0, 0].astype(_f32)
    gbt = gbt_ref[0, 0].astype(_f32)
    gab = gab_ref[0, 0].astype(_f32)
    gbb = gbb_ref[0, 0].astype(_f32)
    b1 = b1_ref[...]
    h_ft = jnp.tanh(gat[:, :F] + gbt[:, F:] + b1)
    h_fb = jnp.tanh(gab[:, :F] + gbb[:, F:] + b1)
    h_bt = jnp.tanh(gbt[:, :F] + gat[:, F:] + b1)
    h_bb = jnp.tanh(gbb[:, :F] + gab[:, F:] + b1)
    x = jnp.concatenate([h_ft, h_fb, h_bt, h_bb], axis=0)
    x = jnp.tanh(jnp.dot(x, w2_ref[...], preferred_element_type=_f32)
                 + b2_ref[...])
    m = jnp.dot(x, w3_ref[...], preferred_element_type=_f32) + b3_ref[...]
    m_ref[0, 0] = jnp.concatenate([m[:BE], m[BE:2 * BE]], axis=1)
    m_ref[0, 1] = jnp.concatenate([m[2 * BE:3 * BE], m[3 * BE:]], axis=1)


def _tc_mlp(g4, b1, w2, b2, w3, b3):
    wspec = pl.BlockSpec((F, F), lambda c, i: (0, 0))
    bspec = pl.BlockSpec((1, F), lambda c, i: (0, 0))
    hb = EPH // (2 * BE)
    espec = lambda d, off: pl.BlockSpec((1, 1, BE, 2 * F),
                                        lambda c, i: (c, d, i + off, 0))
    return pl.pallas_call(
        _mlp_body,
        grid=(N_ENV, EPH // (2 * BE)),
        in_specs=[espec(0, 0), espec(1, 0), espec(0, hb), espec(1, hb),
                  bspec, wspec, bspec, wspec, bspec],
        out_specs=pl.BlockSpec((1, 2, BE, 2 * F), lambda c, i: (c, 0, i, 0)),
        out_shape=jax.ShapeDtypeStruct((N_ENV, 2, EPH // 2, 2 * F), _f32),
        interpret=_INTERPRET,
    )(g4, g4, g4, g4, b1, w2, b2, w3, b3)


def _gru_body(s_ref, nf_ref, wir_ref, wiz_ref, win_ref, whr_ref, whz_ref,
              whn_ref, bi_ref, bh_ref, w1s_ref, w1d_ref,
              nfo_ref, t_ref):
    x = s_ref[0]
    h = nf_ref[0]
    dot = lambda a, w: jnp.dot(a, w[...], preferred_element_type=_f32)
    r = jax.nn.sigmoid(dot(x, wir_ref) + bi_ref[0, 0] + dot(h, whr_ref)
                       + bh_ref[0, 0])
    z = jax.nn.sigmoid(dot(x, wiz_ref) + bi_ref[0, 1] + dot(h, whz_ref)
                       + bh_ref[0, 1])
    n = jnp.tanh(dot(x, win_ref) + bi_ref[0, 2]
                 + r * (dot(h, whn_ref) + bh_ref[0, 2]))
    nf2 = (1.0 - z) * n + z * h
    nfo_ref[0] = nf2
    p = jnp.dot(nf2, w1s_ref[...], preferred_element_type=_f32)
    q = jnp.dot(nf2, w1d_ref[...], preferred_element_type=_f32)
    t_ref[0] = jnp.concatenate([p, q], axis=1).astype(jnp.bfloat16)


def _tc_gru(store, nf, wih3, whh3, bih3, bhh3, w1s, w1d):
    blk = lambda c, i: (c, i, 0)
    wspec = pl.BlockSpec((F, F), lambda c, i: (0, 0))
    bspec = pl.BlockSpec((1, 3, F), lambda c, i: (0, 0, 0))
    return pl.pallas_call(
        _gru_body,
        grid=(N_ENV, N // BN),
        in_specs=[pl.BlockSpec((1, BN, F), blk), pl.BlockSpec((1, BN, F), blk),
                  wspec, wspec, wspec, wspec, wspec, wspec,
                  bspec, bspec, wspec, wspec],
        out_specs=(pl.BlockSpec((1, BN, F), blk),
                   pl.BlockSpec((1, BN, 2 * F), blk)),
        out_shape=(jax.ShapeDtypeStruct((N_ENV, N, F), _f32),
                   jax.ShapeDtypeStruct((N_ENV, N, 2 * F), jnp.bfloat16)),
        interpret=_INTERPRET,
    )(store, nf, *wih3, *whh3, bih3, bhh3, w1s, w1d)


def kernel(node_features, edges, W1, b1, W2, b2, W3, b3, W_ih, b_ih, W_hh,
           b_hh, device=0):
    e0 = edges[0].astype(jnp.int32)
    e1 = edges[1].astype(jnp.int32)
    padw = ((0, EPH - E2),)
    e0g = jnp.pad(e0, padw)
    e1g = jnp.pad(e1, padw)
    eab = jnp.stack([e0g.reshape(EPH // K, K), e1g.reshape(EPH // K, K)])
    def mk_dest(ei):
        p = jnp.pad(ei, padw, constant_values=N)
        return jnp.stack([p[:EPH // 2], p[EPH // 2:]],
                         axis=1).reshape(EPH // K, K)

    ef = mk_dest(e0)
    ebk = mk_dest(e1)

    w1s = W1[:, :F].T
    w1d = W1[:, F:].T
    w2t = W2.T
    w3t = W3.T
    wih3 = tuple(W_ih[i * F:(i + 1) * F].T for i in range(3))
    whh3 = tuple(W_hh[i * F:(i + 1) * F].T for i in range(3))
    bih3 = b_ih.reshape(1, 3, F)
    bhh3 = b_hh.reshape(1, 3, F)
    b1r = b1.reshape(1, F)
    b2r = b2.reshape(1, F)
    b3r = b3.reshape(1, F)
    zeros_n = jnp.zeros((N, F), _f32)

    nf = node_features
    t = _tc_pq(nf, w1s, w1d)
    for _ in range(3):
        g = _sc_gather(t.reshape(N_ENV * N, 2 * F), eab)
        g4 = g.reshape(N_ENV, 2, EPH, 2 * F)
        m = _tc_mlp(g4, b1r, w2t, b2r, w3t, b3r)
        store = _sc_scatter(m.reshape(N_ENV, 2, EPH, F), ef, ebk, zeros_n)
        nf, t = _tc_gru(store, nf, wih3, whh3, bih3, bhh3, w1s, w1d)
    return nf


# speedup vs baseline: 1.0770x; 1.0770x over previous
import functools

import jax
import jax.numpy as jnp
from jax import lax
from jax.experimental import pallas as pl
from jax.experimental.pallas import tpu as pltpu
from jax.experimental.pallas import tpu_sc as plsc

N_ENV = 2
N = 10000
E2 = 160000
EPH = 163840
F = 64
NTILES = 16
K = 128
CPT = EPH // (NTILES * K)
MPT = 160
NPT = 624
BN = 1000
BE = 512

_f32 = jnp.float32
_INTERPRET = False


_sc_mesh = plsc.VectorSubcoreMesh(core_axis_name="c", subcore_axis_name="s",
                                  num_cores=2, num_subcores=NTILES)


@functools.partial(
    pl.kernel,
    out_type=jax.ShapeDtypeStruct((N_ENV * 2 * EPH, 2 * F), _f32),
    mesh=_sc_mesh,
    interpret=_INTERPRET,
    scratch_types=[
        pltpu.VMEM((CPT, K), jnp.int32),
        pltpu.VMEM((CPT, K), jnp.int32),
        pltpu.VMEM((K, 2 * F), _f32),
        pltpu.VMEM((K, 2 * F), _f32),
        pltpu.VMEM((K, 2 * F), _f32),
        pltpu.VMEM((K, 2 * F), _f32),
        pltpu.VMEM((K, 2 * F), _f32),
        pltpu.VMEM((K, 2 * F), _f32),
        pltpu.SemaphoreType.DMA,
        pltpu.SemaphoreType.DMA,
        pltpu.SemaphoreType.DMA,
    ],
)
def _sc_gather(t_hbm, eab_hbm, g_hbm,
               idxa_v, idxb_v, ra0, rb0, ra1, rb1, ra2, rb2,
               semA, semB, semC):
    c = lax.axis_index("c")
    s = lax.axis_index("s")
    row0 = pl.multiple_of(s * CPT, 8)
    pltpu.sync_copy(eab_hbm.at[0, c, pl.ds(row0, CPT)], idxa_v)
    pltpu.sync_copy(eab_hbm.at[1, c, pl.ds(row0, CPT)], idxb_v)

    def issue(j, ra, rb, sem):
        pltpu.async_copy(t_hbm.at[idxa_v.at[j]], ra, sem)
        pltpu.async_copy(t_hbm.at[idxb_v.at[j]], rb, sem)

    def drain(j, ra, rb, sem):
        pltpu.make_async_copy(t_hbm.at[idxa_v.at[j]], ra, sem).wait()
        pltpu.make_async_copy(t_hbm.at[idxb_v.at[j]], rb, sem).wait()

    def store(j, ra, rb):
        out_row = pl.multiple_of(c * 2 * EPH + (row0 + j) * K, 8)
        pltpu.sync_copy(ra, g_hbm.at[pl.ds(out_row, K)])
        pltpu.sync_copy(rb, g_hbm.at[pl.ds(EPH + out_row, K)])

    bufs = ((ra0, rb0, semA), (ra1, rb1, semB), (ra2, rb2, semC))
    issue(0, *bufs[0])
    issue(1, *bufs[1])

    def step(j, ph):
        ra, rb, sem = bufs[ph]
        rn, rbn, semn = bufs[(ph + 2) % 3]

        @pl.when(j < CPT)
        def _():
            drain(j, ra, rb, sem)

            @pl.when(j + 2 < CPT)
            def _():
                issue(j + 2, rn, rbn, semn)

            store(j, ra, rb)

    def body(u, carry):
        j0 = 3 * u
        step(j0, 0)
        step(j0 + 1, 1)
        step(j0 + 2, 2)
        return carry

    lax.fori_loop(0, (CPT + 2) // 3, body, 0)


@functools.partial(
    pl.kernel,
    out_type=jax.ShapeDtypeStruct((N_ENV, N, F), _f32),
    mesh=_sc_mesh,
    interpret=_INTERPRET,
    compiler_params=pltpu.CompilerParams(use_tc_tiling_on_sc=False),
    scratch_types=[
        pltpu.VMEM_SHARED((N + 16, F), _f32),
        pltpu.VMEM((CPT, K), jnp.int32),
        pltpu.VMEM((CPT, K), jnp.int32),
        pltpu.VMEM((K, F), _f32),
        pltpu.VMEM((K, F), _f32),
        pltpu.SemaphoreType.DMA,
        pltpu.SemaphoreType.DMA,
    ],
)
def _sc_scatter(m_hbm, ef_hbm, eb_hbm, zeros_hbm, out_hbm, acc_sh, idxf_v,
                idxb_v, m0_v, m1_v, semA, semB):
    c = lax.axis_index("c")
    s = lax.axis_index("s")
    nrow0 = pl.multiple_of(s * NPT, 8)
    pltpu.sync_copy(zeros_hbm.at[pl.ds(nrow0, NPT)],
                    acc_sh.at[pl.ds(nrow0, NPT)])

    @pl.when(s == NTILES - 1)
    def _():
        tail = pl.multiple_of(NTILES * NPT, 8)
        pltpu.sync_copy(zeros_hbm.at[pl.ds(tail, N - NTILES * NPT)],
                        acc_sh.at[pl.ds(tail, N - NTILES * NPT)])

    row0 = pl.multiple_of(s * CPT, 8)
    pltpu.sync_copy(ef_hbm.at[pl.ds(row0, CPT)], idxf_v)
    pltpu.sync_copy(eb_hbm.at[pl.ds(row0, CPT)], idxb_v)
    plsc.subcore_barrier()

    def run_half(d, idx_v):
        def load(j, mv, sem):
            in_row = pl.multiple_of((row0 + j) * K, 8)
            pltpu.async_copy(m_hbm.at[c, d, pl.ds(in_row, K)], mv, sem)

        def drain(j, mv, sem):
            in_row = pl.multiple_of((row0 + j) * K, 8)
            pltpu.make_async_copy(m_hbm.at[c, d, pl.ds(in_row, K)],
                                  mv, sem).wait()

        load(0, m0_v, semA)

        def body(u, carry):
            j0 = 2 * u
            j1 = j0 + 1
            load(j1, m1_v, semB)
            drain(j0, m0_v, semA)
            pltpu.sync_copy(m0_v, acc_sh.at[idx_v.at[j0]], add=True)

            @pl.when(j1 + 1 < CPT)
            def _():
                load(j1 + 1, m0_v, semA)

            drain(j1, m1_v, semB)
            pltpu.sync_copy(m1_v, acc_sh.at[idx_v.at[j1]], add=True)
            return carry

        lax.fori_loop(0, CPT // 2, body, 0)

    run_half(0, idxf_v)
    run_half(1, idxb_v)
    plsc.subcore_barrier()
    pltpu.sync_copy(acc_sh.at[pl.ds(nrow0, NPT)],
                    out_hbm.at[c, pl.ds(nrow0, NPT)])

    @pl.when(s == NTILES - 1)
    def _():
        tail = pl.multiple_of(NTILES * NPT, 8)
        pltpu.sync_copy(acc_sh.at[pl.ds(tail, N - NTILES * NPT)],
                        out_hbm.at[c, pl.ds(tail, N - NTILES * NPT)])


def _pq_body(nf_ref, w1s_ref, w1d_ref, t_ref):
    x = nf_ref[0]
    p = jnp.dot(x, w1s_ref[...], preferred_element_type=_f32)
    q = jnp.dot(x, w1d_ref[...], preferred_element_type=_f32)
    t_ref[0] = jnp.concatenate([p, q], axis=1)


def _tc_pq(nf, w1s, w1d):
    blk = lambda c, i: (c, i, 0)
    wspec = pl.BlockSpec((F, F), lambda c, i: (0, 0))
    return pl.pallas_call(
        _pq_body,
        grid=(N_ENV, N // BN),
        in_specs=[pl.BlockSpec((1, BN, F), blk), wspec, wspec],
        out_specs=pl.BlockSpec((1, BN, 2 * F), blk),
        out_shape=jax.ShapeDtypeStruct((N_ENV, N, 2 * F), _f32),
        interpret=_INTERPRET,
    )(nf, w1s, w1d)


def _mlp_body(gat_ref, gbt_ref, gab_ref, gbb_ref, b1_ref, w2_ref, b2_ref,
              w3_ref, b3_ref, m_ref):
    gat = gat_ref[0, 0]
    gbt = gbt_ref[0, 0]
    gab = gab_ref[0, 0]
    gbb = gbb_ref[0, 0]
    b1 = b1_ref[...]
    h_ft = jnp.tanh(gat[:, :F] + gbt[:, F:] + b1)
    h_fb = jnp.tanh(gab[:, :F] + gbb[:, F:] + b1)
    h_bt = jnp.tanh(gbt[:, :F] + gat[:, F:] + b1)
    h_bb = jnp.tanh(gbb[:, :F] + gab[:, F:] + b1)
    x = jnp.concatenate([h_ft, h_fb, h_bt, h_bb], axis=0)
    x = jnp.tanh(jnp.dot(x, w2_ref[...], preferred_element_type=_f32)
                 + b2_ref[...])
    m = jnp.dot(x, w3_ref[...], preferred_element_type=_f32) + b3_ref[...]
    m_ref[0, 0] = jnp.concatenate([m[:BE], m[BE:2 * BE]], axis=1)
    m_ref[0, 1] = jnp.concatenate([m[2 * BE:3 * BE], m[3 * BE:]], axis=1)


def _tc_mlp(g4, b1, w2, b2, w3, b3):
    wspec = pl.BlockSpec((F, F), lambda c, i: (0, 0))
    bspec = pl.BlockSpec((1, F), lambda c, i: (0, 0))
    hb = EPH // (2 * BE)
    espec = lambda d, off: pl.BlockSpec((1, 1, BE, 2 * F),
                                        lambda c, i: (c, d, i + off, 0))
    return pl.pallas_call(
        _mlp_body,
        grid=(N_ENV, EPH // (2 * BE)),
        in_specs=[espec(0, 0), espec(1, 0), espec(0, hb), espec(1, hb),
                  bspec, wspec, bspec, wspec, bspec],
        out_specs=pl.BlockSpec((1, 2, BE, 2 * F), lambda c, i: (c, 0, i, 0)),
        out_shape=jax.ShapeDtypeStruct((N_ENV, 2, EPH // 2, 2 * F), _f32),
        interpret=_INTERPRET,
    )(g4, g4, g4, g4, b1, w2, b2, w3, b3)


def _gru_body(s_ref, nf_ref, wir_ref, wiz_ref, win_ref, whr_ref, whz_ref,
              whn_ref, bi_ref, bh_ref, w1s_ref, w1d_ref,
              nfo_ref, t_ref):
    x = s_ref[0]
    h = nf_ref[0]
    dot = lambda a, w: jnp.dot(a, w[...], preferred_element_type=_f32)
    r = jax.nn.sigmoid(dot(x, wir_ref) + bi_ref[0, 0] + dot(h, whr_ref)
                       + bh_ref[0, 0])
    z = jax.nn.sigmoid(dot(x, wiz_ref) + bi_ref[0, 1] + dot(h, whz_ref)
                       + bh_ref[0, 1])
    n = jnp.tanh(dot(x, win_ref) + bi_ref[0, 2]
                 + r * (dot(h, whn_ref) + bh_ref[0, 2]))
    nf2 = (1.0 - z) * n + z * h
    nfo_ref[0] = nf2
    p = jnp.dot(nf2, w1s_ref[...], preferred_element_type=_f32)
    q = jnp.dot(nf2, w1d_ref[...], preferred_element_type=_f32)
    t_ref[0] = jnp.concatenate([p, q], axis=1)


def _tc_gru(store, nf, wih3, whh3, bih3, bhh3, w1s, w1d):
    blk = lambda c, i: (c, i, 0)
    wspec = pl.BlockSpec((F, F), lambda c, i: (0, 0))
    bspec = pl.BlockSpec((1, 3, F), lambda c, i: (0, 0, 0))
    return pl.pallas_call(
        _gru_body,
        grid=(N_ENV, N // BN),
        in_specs=[pl.BlockSpec((1, BN, F), blk), pl.BlockSpec((1, BN, F), blk),
                  wspec, wspec, wspec, wspec, wspec, wspec,
                  bspec, bspec, wspec, wspec],
        out_specs=(pl.BlockSpec((1, BN, F), blk),
                   pl.BlockSpec((1, BN, 2 * F), blk)),
        out_shape=(jax.ShapeDtypeStruct((N_ENV, N, F), _f32),
                   jax.ShapeDtypeStruct((N_ENV, N, 2 * F), _f32)),
        interpret=_INTERPRET,
    )(store, nf, *wih3, *whh3, bih3, bhh3, w1s, w1d)


def kernel(node_features, edges, W1, b1, W2, b2, W3, b3, W_ih, b_ih, W_hh,
           b_hh, device=0):
    e0 = edges[0].astype(jnp.int32)
    e1 = edges[1].astype(jnp.int32)
    padw = ((0, EPH - E2),)
    e0g = jnp.pad(e0, padw)
    e1g = jnp.pad(e1, padw)
    env_bias = (jnp.arange(N_ENV, dtype=jnp.int32) * N)[None, :, None, None]
    eab = jnp.stack([e0g.reshape(EPH // K, K),
                     e1g.reshape(EPH // K, K)])[:, None] + env_bias
    def mk_dest(ei):
        p = jnp.pad(ei, padw, constant_values=N)
        return jnp.stack([p[:EPH // 2], p[EPH // 2:]],
                         axis=1).reshape(EPH // K, K)

    ef = mk_dest(e0)
    ebk = mk_dest(e1)

    w1s = W1[:, :F].T
    w1d = W1[:, F:].T
    w2t = W2.T
    w3t = W3.T
    wih3 = tuple(W_ih[i * F:(i + 1) * F].T for i in range(3))
    whh3 = tuple(W_hh[i * F:(i + 1) * F].T for i in range(3))
    bih3 = b_ih.reshape(1, 3, F)
    bhh3 = b_hh.reshape(1, 3, F)
    b1r = b1.reshape(1, F)
    b2r = b2.reshape(1, F)
    b3r = b3.reshape(1, F)
    zeros_n = jnp.zeros((N, F), _f32)

    nf = node_features
    t = _tc_pq(nf, w1s, w1d)
    for _ in range(3):
        g = _sc_gather(t.reshape(N_ENV * N, 2 * F), eab)
        g4 = g.reshape(N_ENV, 2, EPH, 2 * F)
        m = _tc_mlp(g4, b1r, w2t, b2r, w3t, b3r)
        store = _sc_scatter(m.reshape(N_ENV, 2, EPH, F), ef, ebk, zeros_n)
        nf, t = _tc_gru(store, nf, wih3, whh3, bih3, bhh3, w1s, w1d)
    return nf
